# baseline (speedup 1.0000x reference)
import functools

import jax
import jax.numpy as jnp
from jax import lax
from jax.experimental import pallas as pl
from jax.experimental.pallas import tpu as pltpu
from jax.experimental.pallas import tpu_sc as plsc

BATCH = 16384
FIELDS = 26
D = 32
NC = 2
NS = 16
NW = NC * NS
BT = BATCH // 128
BT_PER_W = BT // NW
NK = BT_PER_W * FIELDS
NBUF = 4


def _sc_gather(idxr, table):
    mesh = plsc.VectorSubcoreMesh(core_axis_name="c", subcore_axis_name="s")

    @functools.partial(
        pl.kernel,
        mesh=mesh,
        out_type=jax.ShapeDtypeStruct((FIELDS, D // 8, BT, 1024), jnp.float32),
        compiler_params=pltpu.CompilerParams(
            use_tc_tiling_on_sc=False, needs_layout_passes=False
        ),
        scratch_types=[
            pltpu.VMEM((BT_PER_W, FIELDS, 128), jnp.int32),
            *[pltpu.VMEM((128, D), jnp.float32) for _ in range(NBUF)],
            *[pltpu.VMEM((D * 128,), jnp.float32) for _ in range(2)],
            *[pltpu.SemaphoreType.DMA for _ in range(NBUF + 2)],
        ],
    )
    def k(idx_hbm, table_hbm, out_hbm, idx_v, *scr):
        g_bufs = scr[:NBUF]
        t_bufs = scr[NBUF:NBUF + 2]
        g_sems = scr[NBUF + 2:2 * NBUF + 2]
        s_sems = scr[2 * NBUF + 2:]
        wid = lax.axis_index("s") * NC + lax.axis_index("c")
        pltpu.sync_copy(idx_hbm.at[pl.ds(wid * BT_PER_W, BT_PER_W)], idx_v)

        for kk in range(NBUF):
            pltpu.async_copy(
                table_hbm.at[idx_v.at[kk // FIELDS, kk % FIELDS]],
                g_bufs[kk], g_sems[kk],
            )

        rows = [lax.iota(jnp.int32, 16) + blk * 16 for blk in range(8)]

        def body(g, carry):
            for b in range(NBUF):
                t = b % 2
                kc = g * NBUF + b
                bti = kc // FIELDS
                f = kc - bti * FIELDS
                bt = wid * BT_PER_W + bti

                pltpu.make_async_copy(
                    table_hbm.at[idx_v.at[0, 0]], g_bufs[b], g_sems[b]
                ).wait()

                @pl.when(kc >= 2)
                def _():
                    for dt in range(D // 8):
                        pltpu.make_async_copy(
                            t_bufs[t].at[pl.ds(dt * 1024, 1024)],
                            out_hbm.at[0, 0, 0],
                            s_sems[t],
                        ).wait()

                def col(d, c2):
                    cols = jnp.full((16,), d, dtype=jnp.int32)
                    for blk in range(8):
                        v = plsc.load_gather(g_bufs[b], [rows[blk], cols])
                        t_bufs[t][pl.ds(d * 128 + blk * 16, 16)] = v
                    return c2

                lax.fori_loop(0, D, col, 0)

                for dt in range(D // 8):
                    pltpu.async_copy(
                        t_bufs[t].at[pl.ds(dt * 1024, 1024)],
                        out_hbm.at[f, dt, bt],
                        s_sems[t],
                    )

                kn = kc + NBUF

                @pl.when(kn < NK)
                def _():
                    btn = kn // FIELDS
                    fn = kn - btn * FIELDS
                    pltpu.async_copy(
                        table_hbm.at[idx_v.at[btn, fn]], g_bufs[b], g_sems[b]
                    )

            return carry

        lax.fori_loop(0, NK // NBUF, body, 0)

        for t in range(2):
            for dt in range(D // 8):
                pltpu.make_async_copy(
                    t_bufs[t].at[pl.ds(dt * 1024, 1024)],
                    out_hbm.at[0, 0, 0],
                    s_sems[t],
                ).wait()

    return k(idxr, table)


def kernel(inputs, embeddings):
    idxr = inputs.reshape(BT, 128, FIELDS).transpose(0, 2, 1)
    y = _sc_gather(idxr, embeddings)
    out = y.reshape(FIELDS, D // 8, BT, 8, 128).transpose(2, 4, 0, 1, 3)
    return out.reshape(BATCH, FIELDS, D)


# speedup vs baseline: 1.1112x; 1.1112x over previous
import functools

import jax
import jax.numpy as jnp
from jax import lax
from jax.experimental import pallas as pl
from jax.experimental.pallas import tpu as pltpu
from jax.experimental.pallas import tpu_sc as plsc

BATCH = 16384
FIELDS = 26
D = 32
NC = 2
NS = 16
NW = NC * NS
BT = BATCH // 128
BT_PER_W = BT // NW
NK = BT_PER_W * FIELDS
NBUF = 4


def _sc_gather(idxr, table):
    mesh = plsc.VectorSubcoreMesh(core_axis_name="c", subcore_axis_name="s")

    @functools.partial(
        pl.kernel,
        mesh=mesh,
        out_type=jax.ShapeDtypeStruct((FIELDS, D // 8, BT, 1024), jnp.float32),
        compiler_params=pltpu.CompilerParams(
            use_tc_tiling_on_sc=False, needs_layout_passes=False
        ),
        scratch_types=[
            pltpu.VMEM((BT_PER_W, FIELDS, 128), jnp.int32),
            *[pltpu.VMEM((128, D), jnp.float32) for _ in range(NBUF)],
            *[pltpu.VMEM((D * 128,), jnp.float32) for _ in range(2)],
            *[pltpu.SemaphoreType.DMA for _ in range(NBUF + 2)],
        ],
    )
    def k(idx_hbm, table_hbm, out_hbm, idx_v, *scr):
        g_bufs = scr[:NBUF]
        t_bufs = scr[NBUF:NBUF + 2]
        g_sems = scr[NBUF + 2:2 * NBUF + 2]
        s_sems = scr[2 * NBUF + 2:]
        wid = lax.axis_index("s") * NC + lax.axis_index("c")
        pltpu.sync_copy(idx_hbm.at[pl.ds(wid * BT_PER_W, BT_PER_W)], idx_v)

        for kk in range(NBUF):
            pltpu.async_copy(
                table_hbm.at[idx_v.at[kk // FIELDS, kk % FIELDS]],
                g_bufs[kk], g_sems[kk],
            )

        rows = [lax.iota(jnp.int32, 16) + blk * 16 for blk in range(8)]

        def body(g, carry):
            for b in range(NBUF):
                t = b % 2
                kc = g * NBUF + b
                bti = kc // FIELDS
                f = kc - bti * FIELDS
                bt = wid * BT_PER_W + bti

                pltpu.make_async_copy(
                    table_hbm.at[idx_v.at[0, 0]], g_bufs[b], g_sems[b]
                ).wait()

                @pl.when(kc >= 2)
                def _():
                    for dt in range(D // 8):
                        pltpu.make_async_copy(
                            t_bufs[t].at[pl.ds(dt * 1024, 1024)],
                            out_hbm.at[0, 0, 0],
                            s_sems[t],
                        ).wait()

                @plsc.parallel_loop(0, D, unroll=4)
                def col(d):
                    cols = jnp.full((16,), d, dtype=jnp.int32)
                    vs = [
                        plsc.load_gather(g_bufs[b], [rows[blk], cols])
                        for blk in range(8)
                    ]
                    for blk in range(8):
                        t_bufs[t][pl.ds(d * 128 + blk * 16, 16)] = vs[blk]

                for dt in range(D // 8):
                    pltpu.async_copy(
                        t_bufs[t].at[pl.ds(dt * 1024, 1024)],
                        out_hbm.at[f, dt, bt],
                        s_sems[t],
                    )

                kn = kc + NBUF

                @pl.when(kn < NK)
                def _():
                    btn = kn // FIELDS
                    fn = kn - btn * FIELDS
                    pltpu.async_copy(
                        table_hbm.at[idx_v.at[btn, fn]], g_bufs[b], g_sems[b]
                    )

            return carry

        lax.fori_loop(0, NK // NBUF, body, 0)

        for t in range(2):
            for dt in range(D // 8):
                pltpu.make_async_copy(
                    t_bufs[t].at[pl.ds(dt * 1024, 1024)],
                    out_hbm.at[0, 0, 0],
                    s_sems[t],
                ).wait()

    return k(idxr, table)


def kernel(inputs, embeddings):
    idxr = inputs.reshape(BT, 128, FIELDS).transpose(0, 2, 1)
    y = _sc_gather(idxr, embeddings)
    out = y.reshape(FIELDS, D // 8, BT, 8, 128).transpose(2, 4, 0, 1, 3)
    return out.reshape(BATCH, FIELDS, D)


# speedup vs baseline: 1.2707x; 1.1436x over previous
import functools

import jax
import jax.numpy as jnp
from jax import lax
from jax.experimental import pallas as pl
from jax.experimental.pallas import tpu as pltpu
from jax.experimental.pallas import tpu_sc as plsc

VOCAB = 1000000
BATCH = 16384
FIELDS = 26
D = 32
NC = 2
NS = 16
NW = NC * NS
BT = BATCH // 128
BT_PER_W = BT // NW
NK = BT_PER_W * FIELDS
NBUF = 4
NBLK = VOCAB // 128
REM = VOCAB - NBLK * 128
RM_ROWS = VOCAB // 4


def _sc_transpose(tbl_t):
    mesh = plsc.VectorSubcoreMesh(core_axis_name="c", subcore_axis_name="s")

    @functools.partial(
        pl.kernel,
        mesh=mesh,
        out_type=jax.ShapeDtypeStruct((RM_ROWS, 128), jnp.float32),
        compiler_params=pltpu.CompilerParams(
            use_tc_tiling_on_sc=True, needs_layout_passes=False,
            disable_bounds_checks=True,
        ),
        scratch_types=[
            *[pltpu.VMEM((D, 128), jnp.float32) for _ in range(4)],
            *[pltpu.SemaphoreType.DMA for _ in range(4)],
        ],
    )
    def k(tbl_hbm, rm_hbm, s0, s1, t0, t1, ls0, ls1, ss0, ss1):
        s_bufs = (s0, s1)
        t_bufs = (t0, t1)
        l_sems = (ls0, ls1)
        s_sems = (ss0, ss1)
        wid = lax.axis_index("s") * NC + lax.axis_index("c")
        nb = NBLK // NW
        ext = NBLK - nb * NW
        lo = wid * nb + jnp.minimum(wid, ext)
        n = nb + jnp.where(wid < ext, 1, 0)

        for p in range(2):

            @pl.when(p < n)
            def _():
                pltpu.async_copy(
                    tbl_hbm.at[:, pl.ds((lo + p) * 128, 128)],
                    s_bufs[p], l_sems[p],
                )

        def body(i, carry):
            for p in range(2):

                @pl.when(i % 2 == p)
                def _():
                    j = lo + i
                    pltpu.make_async_copy(
                        tbl_hbm.at[:, pl.ds(0, 128)], s_bufs[p], l_sems[p]
                    ).wait()

                    @pl.when(i >= 2)
                    def _():
                        pltpu.make_async_copy(
                            t_bufs[p], rm_hbm.at[pl.ds(0, D)], s_sems[p]
                        ).wait()

                    @plsc.parallel_loop(0, D, unroll=4)
                    def col(c):
                        vs = []
                        for lb in range(8):
                            rows = lax.iota(jnp.int32, 16) + (lb % 2) * 16
                            cols = jnp.full((16,), c * 4 + lb // 2, jnp.int32)
                            vs.append(plsc.load_gather(s_bufs[p], [rows, cols]))
                        for lb in range(8):
                            t_bufs[p][c, pl.ds(lb * 16, 16)] = vs[lb]

                    pltpu.async_copy(
                        t_bufs[p], rm_hbm.at[pl.ds(j * D, D)], s_sems[p]
                    )

                    @pl.when(i + 2 < n)
                    def _():
                        pltpu.async_copy(
                            tbl_hbm.at[:, pl.ds((j + 2) * 128, 128)],
                            s_bufs[p], l_sems[p],
                        )

            return carry

        lax.fori_loop(0, n, body, 0)

        for p in range(2):

            @pl.when(n >= p + 1)
            def _():
                pltpu.make_async_copy(
                    t_bufs[p], rm_hbm.at[pl.ds(0, D)], s_sems[p]
                ).wait()

        @pl.when(wid == 0)
        def _():
            jr = (wid + NBLK) * 128
            pltpu.sync_copy(tbl_hbm.at[:, pl.ds(jr, 128)], s_bufs[0])

            @plsc.parallel_loop(0, REM // 4, unroll=4)
            def rcol(c):
                vs = []
                for lb in range(8):
                    rows = lax.iota(jnp.int32, 16) + (lb % 2) * 16
                    cols = jnp.full((16,), c * 4 + lb // 2, jnp.int32)
                    vs.append(plsc.load_gather(s_bufs[0], [rows, cols]))
                for lb in range(8):
                    t_bufs[0][c, pl.ds(lb * 16, 16)] = vs[lb]

            pltpu.sync_copy(
                t_bufs[0].at[pl.ds(0, REM // 4)],
                rm_hbm.at[pl.ds(NBLK * D, REM // 4)],
            )

    return k(tbl_t)


def _sc_gather(idxr, rm):
    mesh = plsc.VectorSubcoreMesh(core_axis_name="c", subcore_axis_name="s")

    @functools.partial(
        pl.kernel,
        mesh=mesh,
        out_type=jax.ShapeDtypeStruct((FIELDS * (D // 8) * BT * 8, 128),
                                      jnp.float32),
        compiler_params=pltpu.CompilerParams(
            use_tc_tiling_on_sc=True, needs_layout_passes=False
        ),
        scratch_types=[
            pltpu.VMEM((NK, 128), jnp.int32),
            pltpu.VMEM((NK, 128), jnp.int32),
            *[pltpu.VMEM((128, 128), jnp.float32) for _ in range(NBUF)],
            *[pltpu.VMEM((D, 128), jnp.float32) for _ in range(2)],
            *[pltpu.SemaphoreType.DMA for _ in range(NBUF + 2)],
        ],
    )
    def k(idx_hbm, rm_hbm, out_hbm, idx_v, rem_v, *scr):
        g_bufs = scr[:NBUF]
        t_bufs = scr[NBUF:NBUF + 2]
        g_sems = scr[NBUF + 2:2 * NBUF + 2]
        s_sems = scr[2 * NBUF + 2:]
        wid = lax.axis_index("s") * NC + lax.axis_index("c")
        pltpu.sync_copy(idx_hbm.at[pl.ds(wid * NK, NK)], idx_v)

        @plsc.parallel_loop(0, NK * 8, unroll=4)
        def split(z):
            kc = z // 8
            blk = z - kc * 8
            q = idx_v[kc, pl.ds(blk * 16, 16)]
            rem_v[kc, pl.ds(blk * 16, 16)] = (q & 3) * 32
            idx_v[kc, pl.ds(blk * 16, 16)] = lax.shift_right_logical(q, 2)

        for kk in range(NBUF):
            pltpu.async_copy(rm_hbm.at[idx_v.at[kk]], g_bufs[kk], g_sems[kk])

        rows = [lax.iota(jnp.int32, 16) + blk * 16 for blk in range(8)]

        def body(g, carry):
            for b in range(NBUF):
                t = b % 2
                kc = g * NBUF + b
                bti = kc // FIELDS
                f = kc - bti * FIELDS
                bt = wid * BT_PER_W + bti

                pltpu.make_async_copy(
                    rm_hbm.at[idx_v.at[0]], g_bufs[b], g_sems[b]
                ).wait()

                @pl.when(kc >= 2)
                def _():
                    for dt in range(D // 8):
                        pltpu.make_async_copy(
                            t_bufs[t].at[pl.ds(dt * 8, 8)],
                            out_hbm.at[pl.ds(0, 8)],
                            s_sems[t],
                        ).wait()

                for blk in range(8):
                    remv = rem_v[kc, pl.ds(blk * 16, 16)]

                    @plsc.parallel_loop(0, D, unroll=4)
                    def col(d):
                        v = plsc.load_gather(g_bufs[b], [rows[blk], remv + d])
                        t_bufs[t][d, pl.ds(blk * 16, 16)] = v

                row0 = ((f * (D // 8)) * BT + bt) * 8
                for dt in range(D // 8):
                    pltpu.async_copy(
                        t_bufs[t].at[pl.ds(dt * 8, 8)],
                        out_hbm.at[pl.ds(row0 + dt * BT * 8, 8)],
                        s_sems[t],
                    )

                kn = kc + NBUF

                @pl.when(kn < NK)
                def _():
                    pltpu.async_copy(
                        rm_hbm.at[idx_v.at[kn]], g_bufs[b], g_sems[b]
                    )

            return carry

        lax.fori_loop(0, NK // NBUF, body, 0)

        for t in range(2):
            for dt in range(D // 8):
                pltpu.make_async_copy(
                    t_bufs[t].at[pl.ds(dt * 8, 8)],
                    out_hbm.at[pl.ds(0, 8)],
                    s_sems[t],
                ).wait()

    return k(idxr, rm)


def kernel(inputs, embeddings):
    rm = _sc_transpose(embeddings.T)
    idxr = (
        inputs.reshape(BT, 128, FIELDS)
        .transpose(0, 2, 1)
        .reshape(BT * FIELDS, 128)
    )
    y = _sc_gather(idxr, rm)
    out = y.reshape(FIELDS, D // 8, BT, 8, 128).transpose(2, 4, 0, 1, 3)
    return out.reshape(BATCH, FIELDS, D)


# speedup vs baseline: 1.5618x; 1.2290x over previous
import functools

import jax
import jax.numpy as jnp
from jax import lax
from jax.experimental import pallas as pl
from jax.experimental.pallas import tpu as pltpu
from jax.experimental.pallas import tpu_sc as plsc

VOCAB = 1000000
BATCH = 16384
FIELDS = 26
D = 32
NC = 2
NS = 16
NW = NC * NS
BT = BATCH // 128
BT_PER_W = BT // NW
NK = BT_PER_W * FIELDS
NBUF = 4
NBLK = VOCAB // 128
REM = VOCAB - NBLK * 128
RM_ROWS = VOCAB // 4


def _sc_transpose(tbl_t):
    mesh = plsc.VectorSubcoreMesh(core_axis_name="c", subcore_axis_name="s")

    @functools.partial(
        pl.kernel,
        mesh=mesh,
        out_type=jax.ShapeDtypeStruct((RM_ROWS, 128), jnp.float32),
        compiler_params=pltpu.CompilerParams(
            use_tc_tiling_on_sc=True, needs_layout_passes=False,
            disable_bounds_checks=True,
        ),
        scratch_types=[
            *[pltpu.VMEM((D, 128), jnp.float32) for _ in range(4)],
            *[pltpu.SemaphoreType.DMA for _ in range(4)],
        ],
    )
    def k(tbl_hbm, rm_hbm, s0, s1, t0, t1, ls0, ls1, ss0, ss1):
        s_bufs = (s0, s1)
        t_bufs = (t0, t1)
        l_sems = (ls0, ls1)
        s_sems = (ss0, ss1)
        wid = lax.axis_index("s") * NC + lax.axis_index("c")
        nb = NBLK // NW
        ext = NBLK - nb * NW
        lo = wid * nb + jnp.minimum(wid, ext)
        n = nb + jnp.where(wid < ext, 1, 0)

        for p in range(2):

            @pl.when(p < n)
            def _():
                pltpu.async_copy(
                    tbl_hbm.at[:, pl.ds((lo + p) * 128, 128)],
                    s_bufs[p], l_sems[p],
                )

        def body(i, carry):
            for p in range(2):

                @pl.when(i % 2 == p)
                def _():
                    j = lo + i
                    pltpu.make_async_copy(
                        tbl_hbm.at[:, pl.ds(0, 128)], s_bufs[p], l_sems[p]
                    ).wait()

                    @pl.when(i >= 2)
                    def _():
                        pltpu.make_async_copy(
                            t_bufs[p], rm_hbm.at[pl.ds(0, D)], s_sems[p]
                        ).wait()

                    @plsc.parallel_loop(0, D, unroll=4)
                    def col(c):
                        vs = []
                        for lb in range(8):
                            vb = (lax.iota(jnp.int32, 16) + lb * 16 + (c & 15)) & 127
                            vs.append(plsc.load_gather(
                                s_bufs[p],
                                [vb & 31, lax.shift_right_logical(vb, 5) + c * 4],
                            ))
                        for lb in range(8):
                            t_bufs[p][c, pl.ds(lb * 16, 16)] = vs[lb]

                    pltpu.async_copy(
                        t_bufs[p], rm_hbm.at[pl.ds(j * D, D)], s_sems[p]
                    )

                    @pl.when(i + 2 < n)
                    def _():
                        pltpu.async_copy(
                            tbl_hbm.at[:, pl.ds((j + 2) * 128, 128)],
                            s_bufs[p], l_sems[p],
                        )

            return carry

        lax.fori_loop(0, n, body, 0)

        for p in range(2):

            @pl.when(n >= p + 1)
            def _():
                pltpu.make_async_copy(
                    t_bufs[p], rm_hbm.at[pl.ds(0, D)], s_sems[p]
                ).wait()

        @pl.when(wid == 0)
        def _():
            jr = (wid + NBLK) * 128
            pltpu.sync_copy(tbl_hbm.at[:, pl.ds(jr, 128)], s_bufs[0])

            @plsc.parallel_loop(0, REM // 4, unroll=4)
            def rcol(c):
                vs = []
                for lb in range(8):
                    vb = (lax.iota(jnp.int32, 16) + lb * 16 + (c & 15)) & 127
                    vs.append(plsc.load_gather(
                        s_bufs[0],
                        [vb & 31, lax.shift_right_logical(vb, 5) + c * 4],
                    ))
                for lb in range(8):
                    t_bufs[0][c, pl.ds(lb * 16, 16)] = vs[lb]

            pltpu.sync_copy(
                t_bufs[0].at[pl.ds(0, REM // 4)],
                rm_hbm.at[pl.ds(NBLK * D, REM // 4)],
            )

    return k(tbl_t)


def _sc_gather(idxr, rm):
    mesh = plsc.VectorSubcoreMesh(core_axis_name="c", subcore_axis_name="s")

    @functools.partial(
        pl.kernel,
        mesh=mesh,
        out_type=jax.ShapeDtypeStruct((FIELDS * (D // 8) * BT * 8, 128),
                                      jnp.float32),
        compiler_params=pltpu.CompilerParams(
            use_tc_tiling_on_sc=True, needs_layout_passes=False
        ),
        scratch_types=[
            pltpu.VMEM((NK, 128), jnp.int32),
            pltpu.VMEM((NK, 128), jnp.int32),
            *[pltpu.VMEM((128, 128), jnp.float32) for _ in range(NBUF)],
            *[pltpu.VMEM((D, 128), jnp.float32) for _ in range(2)],
            *[pltpu.SemaphoreType.DMA for _ in range(NBUF + 2)],
        ],
    )
    def k(idx_hbm, rm_hbm, out_hbm, idx_v, rem_v, *scr):
        g_bufs = scr[:NBUF]
        t_bufs = scr[NBUF:NBUF + 2]
        g_sems = scr[NBUF + 2:2 * NBUF + 2]
        s_sems = scr[2 * NBUF + 2:]
        wid = lax.axis_index("s") * NC + lax.axis_index("c")
        pltpu.sync_copy(idx_hbm.at[pl.ds(wid * NK, NK)], idx_v)

        @plsc.parallel_loop(0, NK * 8, unroll=4)
        def split(z):
            kc = z // 8
            blk = z - kc * 8
            q = idx_v[kc, pl.ds(blk * 16, 16)]
            line = lax.shift_right_logical(q, 2)
            rem_v[kc, pl.ds(blk * 16, 16)] = (q & 3) * 32 - (line & 15)
            idx_v[kc, pl.ds(blk * 16, 16)] = line

        for kk in range(NBUF):
            pltpu.async_copy(rm_hbm.at[idx_v.at[kk]], g_bufs[kk], g_sems[kk])

        rows = [lax.iota(jnp.int32, 16) + blk * 16 for blk in range(8)]

        def body(g, carry):
            for b in range(NBUF):
                t = b % 2
                kc = g * NBUF + b
                bti = kc // FIELDS
                f = kc - bti * FIELDS
                bt = wid * BT_PER_W + bti

                pltpu.make_async_copy(
                    rm_hbm.at[idx_v.at[0]], g_bufs[b], g_sems[b]
                ).wait()

                @pl.when(kc >= 2)
                def _():
                    for dt in range(D // 8):
                        pltpu.make_async_copy(
                            t_bufs[t].at[pl.ds(dt * 8, 8)],
                            out_hbm.at[pl.ds(0, 8)],
                            s_sems[t],
                        ).wait()

                for blk in range(8):
                    remv = rem_v[kc, pl.ds(blk * 16, 16)]

                    @plsc.parallel_loop(0, D, unroll=4)
                    def col(d):
                        v = plsc.load_gather(
                            g_bufs[b], [rows[blk], (remv + d) & 127]
                        )
                        t_bufs[t][d, pl.ds(blk * 16, 16)] = v

                row0 = ((f * (D // 8)) * BT + bt) * 8
                for dt in range(D // 8):
                    pltpu.async_copy(
                        t_bufs[t].at[pl.ds(dt * 8, 8)],
                        out_hbm.at[pl.ds(row0 + dt * BT * 8, 8)],
                        s_sems[t],
                    )

                kn = kc + NBUF

                @pl.when(kn < NK)
                def _():
                    pltpu.async_copy(
                        rm_hbm.at[idx_v.at[kn]], g_bufs[b], g_sems[b]
                    )

            return carry

        lax.fori_loop(0, NK // NBUF, body, 0)

        for t in range(2):
            for dt in range(D // 8):
                pltpu.make_async_copy(
                    t_bufs[t].at[pl.ds(dt * 8, 8)],
                    out_hbm.at[pl.ds(0, 8)],
                    s_sems[t],
                ).wait()

    return k(idxr, rm)


def kernel(inputs, embeddings):
    rm = _sc_transpose(embeddings.T)
    idxr = (
        inputs.reshape(BT, 128, FIELDS)
        .transpose(0, 2, 1)
        .reshape(BT * FIELDS, 128)
    )
    y = _sc_gather(idxr, rm)
    out = y.reshape(FIELDS, D // 8, BT, 8, 128).transpose(2, 4, 0, 1, 3)
    return out.reshape(BATCH, FIELDS, D)


# speedup vs baseline: 3.0573x; 1.9576x over previous
import functools

import jax
import jax.numpy as jnp
from jax import lax
from jax.experimental import pallas as pl
from jax.experimental.pallas import tpu as pltpu
from jax.experimental.pallas import tpu_sc as plsc

VOCAB = 1000000
BATCH = 16384
FIELDS = 26
D = 32
NC = 2
NS = 16
NW = NC * NS
BT = BATCH // 128
BT_PER_W = BT // NW
NK = BT_PER_W * FIELDS
NBUF = 4
NBLK = VOCAB // 128
REM = VOCAB - NBLK * 128
RM_ROWS = VOCAB // 4


def _sc_transpose(tbl_t):
    mesh = plsc.VectorSubcoreMesh(core_axis_name="c", subcore_axis_name="s")

    @functools.partial(
        pl.kernel,
        mesh=mesh,
        out_type=jax.ShapeDtypeStruct((RM_ROWS, 128), jnp.float32),
        compiler_params=pltpu.CompilerParams(
            use_tc_tiling_on_sc=True, needs_layout_passes=False,
            disable_bounds_checks=True,
        ),
        scratch_types=[
            *[pltpu.VMEM((D, 128), jnp.float32) for _ in range(4)],
            *[pltpu.SemaphoreType.DMA for _ in range(4)],
        ],
    )
    def k(tbl_hbm, rm_hbm, s0, s1, t0, t1, ls0, ls1, ss0, ss1):
        s_bufs = (s0, s1)
        t_bufs = (t0, t1)
        l_sems = (ls0, ls1)
        s_sems = (ss0, ss1)
        wid = lax.axis_index("s") * NC + lax.axis_index("c")
        nb = NBLK // NW
        ext = NBLK - nb * NW
        lo = wid * nb + jnp.minimum(wid, ext)
        n = nb + jnp.where(wid < ext, 1, 0)

        iot = lax.iota(jnp.int32, 16)
        crow = [lax.shift_right_logical(iot + g * 16, 2) for g in range(8)]
        pbase = [((iot + g * 16) & 3) - crow[g] * 6 for g in range(8)]

        for p in range(2):

            @pl.when(p < n)
            def _():
                pltpu.async_copy(
                    tbl_hbm.at[:, pl.ds((lo + p) * 128, 128)],
                    s_bufs[p], l_sems[p],
                )

        def body(i, carry):
            for p in range(2):

                @pl.when(i % 2 == p)
                def _():
                    j = lo + i
                    pltpu.make_async_copy(
                        tbl_hbm.at[:, pl.ds(0, 128)], s_bufs[p], l_sems[p]
                    ).wait()

                    @pl.when(i >= 2)
                    def _():
                        pltpu.make_async_copy(
                            t_bufs[p], rm_hbm.at[pl.ds(0, D)], s_sems[p]
                        ).wait()

                    @plsc.parallel_loop(0, D, unroll=4)
                    def col(d):
                        vals = [
                            s_bufs[p][d, pl.ds(g * 16, 16)] for g in range(8)
                        ]
                        for g in range(8):
                            pv = (pbase[g] + d * 4) & 127
                            plsc.store_scatter(
                                t_bufs[p], [crow[g], pv], vals[g]
                            )

                    pltpu.async_copy(
                        t_bufs[p], rm_hbm.at[pl.ds(j * D, D)], s_sems[p]
                    )

                    @pl.when(i + 2 < n)
                    def _():
                        pltpu.async_copy(
                            tbl_hbm.at[:, pl.ds((j + 2) * 128, 128)],
                            s_bufs[p], l_sems[p],
                        )

            return carry

        lax.fori_loop(0, n, body, 0)

        for p in range(2):

            @pl.when(n >= p + 1)
            def _():
                pltpu.make_async_copy(
                    t_bufs[p], rm_hbm.at[pl.ds(0, D)], s_sems[p]
                ).wait()

        @pl.when(wid == 0)
        def _():
            jr = (wid + NBLK) * 128
            pltpu.sync_copy(tbl_hbm.at[:, pl.ds(jr, 128)], s_bufs[0])

            @plsc.parallel_loop(0, D, unroll=4)
            def rcol(d):
                vals = [s_bufs[0][d, pl.ds(g * 16, 16)] for g in range(4)]
                for g in range(4):
                    pv = (pbase[g] + d * 4) & 127
                    plsc.store_scatter(t_bufs[0], [crow[g], pv], vals[g])

            pltpu.sync_copy(
                t_bufs[0].at[pl.ds(0, REM // 4)],
                rm_hbm.at[pl.ds(NBLK * D, REM // 4)],
            )

    return k(tbl_t)


def _sc_gather(idxr, rm):
    mesh = plsc.VectorSubcoreMesh(core_axis_name="c", subcore_axis_name="s")

    @functools.partial(
        pl.kernel,
        mesh=mesh,
        out_type=jax.ShapeDtypeStruct((FIELDS * (D // 8) * BT * 8, 128),
                                      jnp.float32),
        compiler_params=pltpu.CompilerParams(
            use_tc_tiling_on_sc=True, needs_layout_passes=False
        ),
        scratch_types=[
            pltpu.VMEM((NK, 128), jnp.int32),
            pltpu.VMEM((NK, 128), jnp.int32),
            *[pltpu.VMEM((128, 128), jnp.float32) for _ in range(NBUF)],
            *[pltpu.VMEM((D, 128), jnp.float32) for _ in range(2)],
            *[pltpu.SemaphoreType.DMA for _ in range(NBUF + 2)],
        ],
    )
    def k(idx_hbm, rm_hbm, out_hbm, idx_v, rem_v, *scr):
        g_bufs = scr[:NBUF]
        t_bufs = scr[NBUF:NBUF + 2]
        g_sems = scr[NBUF + 2:2 * NBUF + 2]
        s_sems = scr[2 * NBUF + 2:]
        wid = lax.axis_index("s") * NC + lax.axis_index("c")
        pltpu.sync_copy(idx_hbm.at[pl.ds(wid * NK, NK)], idx_v)

        @plsc.parallel_loop(0, NK * 8, unroll=4)
        def split(z):
            kc = z // 8
            blk = z - kc * 8
            q = idx_v[kc, pl.ds(blk * 16, 16)]
            line = lax.shift_right_logical(q, 2)
            rem_v[kc, pl.ds(blk * 16, 16)] = (q & 3) - (line & 31) * 6
            idx_v[kc, pl.ds(blk * 16, 16)] = line

        for kk in range(NBUF):
            pltpu.async_copy(rm_hbm.at[idx_v.at[kk]], g_bufs[kk], g_sems[kk])

        rows = [lax.iota(jnp.int32, 16) + blk * 16 for blk in range(8)]

        def body(g, carry):
            for b in range(NBUF):
                t = b % 2
                kc = g * NBUF + b
                bti = kc // FIELDS
                f = kc - bti * FIELDS
                bt = wid * BT_PER_W + bti

                pltpu.make_async_copy(
                    rm_hbm.at[idx_v.at[0]], g_bufs[b], g_sems[b]
                ).wait()

                @pl.when(kc >= 2)
                def _():
                    for dt in range(D // 8):
                        pltpu.make_async_copy(
                            t_bufs[t].at[pl.ds(dt * 8, 8)],
                            out_hbm.at[pl.ds(0, 8)],
                            s_sems[t],
                        ).wait()

                for blk in range(8):
                    remv = rem_v[kc, pl.ds(blk * 16, 16)]

                    @plsc.parallel_loop(0, D, unroll=4)
                    def col(d):
                        v = plsc.load_gather(
                            g_bufs[b], [rows[blk], (remv + d * 4) & 127]
                        )
                        t_bufs[t][d, pl.ds(blk * 16, 16)] = v

                row0 = ((f * (D // 8)) * BT + bt) * 8
                for dt in range(D // 8):
                    pltpu.async_copy(
                        t_bufs[t].at[pl.ds(dt * 8, 8)],
                        out_hbm.at[pl.ds(row0 + dt * BT * 8, 8)],
                        s_sems[t],
                    )

                kn = kc + NBUF

                @pl.when(kn < NK)
                def _():
                    pltpu.async_copy(
                        rm_hbm.at[idx_v.at[kn]], g_bufs[b], g_sems[b]
                    )

            return carry

        lax.fori_loop(0, NK // NBUF, body, 0)

        for t in range(2):
            for dt in range(D // 8):
                pltpu.make_async_copy(
                    t_bufs[t].at[pl.ds(dt * 8, 8)],
                    out_hbm.at[pl.ds(0, 8)],
                    s_sems[t],
                ).wait()

    return k(idxr, rm)


def kernel(inputs, embeddings):
    rm = _sc_transpose(embeddings.T)
    idxr = (
        inputs.reshape(BT, 128, FIELDS)
        .transpose(0, 2, 1)
        .reshape(BT * FIELDS, 128)
    )
    y = _sc_gather(idxr, rm)
    out = y.reshape(FIELDS, D // 8, BT, 8, 128).transpose(2, 4, 0, 1, 3)
    return out.reshape(BATCH, FIELDS, D)


# speedup vs baseline: 3.7760x; 1.2351x over previous
import functools

import jax
import jax.numpy as jnp
from jax import lax
from jax.experimental import pallas as pl
from jax.experimental.pallas import tpu as pltpu
from jax.experimental.pallas import tpu_sc as plsc

VOCAB = 1000000
BATCH = 16384
FIELDS = 26
D = 32
NC = 2
NS = 16
NW = NC * NS
BT = BATCH // 128
BT_PER_W = BT // NW
NK = BT_PER_W * FIELDS
NBUF = 4
NBLK = VOCAB // 128
REM = VOCAB - NBLK * 128
RM_ROWS = VOCAB // 4


def _sc_transpose(tbl_t):
    mesh = plsc.VectorSubcoreMesh(core_axis_name="c", subcore_axis_name="s")

    @functools.partial(
        pl.kernel,
        mesh=mesh,
        out_type=jax.ShapeDtypeStruct((RM_ROWS, 128), jnp.float32),
        compiler_params=pltpu.CompilerParams(
            use_tc_tiling_on_sc=True, needs_layout_passes=False,
            disable_bounds_checks=True,
        ),
        scratch_types=[
            *[pltpu.VMEM((D, 128), jnp.float32) for _ in range(4)],
            *[pltpu.SemaphoreType.DMA for _ in range(4)],
        ],
    )
    def k(tbl_hbm, rm_hbm, s0, s1, t0, t1, ls0, ls1, ss0, ss1):
        s_bufs = (s0, s1)
        t_bufs = (t0, t1)
        l_sems = (ls0, ls1)
        s_sems = (ss0, ss1)
        wid = lax.axis_index("s") * NC + lax.axis_index("c")
        nb = NBLK // NW
        ext = NBLK - nb * NW
        lo = wid * nb + jnp.minimum(wid, ext)
        n = nb + jnp.where(wid < ext, 1, 0)

        iot = lax.iota(jnp.int32, 16)
        uvec = [iot + g * 16 for g in range(8)]
        crow = [lax.shift_right_logical(uvec[g], 2) for g in range(8)]
        cbase = [(uvec[g] & 3) * 32 for g in range(8)]

        for p in range(2):

            @pl.when(p < n)
            def _():
                pltpu.async_copy(
                    tbl_hbm.at[:, pl.ds((lo + p) * 128, 128)],
                    s_bufs[p], l_sems[p],
                )

        def body(i, carry):
            for p in range(2):

                @pl.when(i % 2 == p)
                def _():
                    j = lo + i
                    pltpu.make_async_copy(
                        tbl_hbm.at[:, pl.ds(0, 128)], s_bufs[p], l_sems[p]
                    ).wait()

                    @pl.when(i >= 2)
                    def _():
                        pltpu.make_async_copy(
                            t_bufs[p], rm_hbm.at[pl.ds(0, D)], s_sems[p]
                        ).wait()

                    @plsc.parallel_loop(0, D, unroll=4)
                    def col(d):
                        vals = [
                            s_bufs[p][d, pl.ds(g * 16, 16)] for g in range(8)
                        ]
                        for g in range(8):
                            pv = cbase[g] + ((d - uvec[g]) & 31)
                            plsc.store_scatter(
                                t_bufs[p], [crow[g], pv], vals[g]
                            )

                    pltpu.async_copy(
                        t_bufs[p], rm_hbm.at[pl.ds(j * D, D)], s_sems[p]
                    )

                    @pl.when(i + 2 < n)
                    def _():
                        pltpu.async_copy(
                            tbl_hbm.at[:, pl.ds((j + 2) * 128, 128)],
                            s_bufs[p], l_sems[p],
                        )

            return carry

        lax.fori_loop(0, n, body, 0)

        for p in range(2):

            @pl.when(n >= p + 1)
            def _():
                pltpu.make_async_copy(
                    t_bufs[p], rm_hbm.at[pl.ds(0, D)], s_sems[p]
                ).wait()

        @pl.when(wid == 0)
        def _():
            jr = (wid + NBLK) * 128
            pltpu.sync_copy(tbl_hbm.at[:, pl.ds(jr, 128)], s_bufs[0])

            @plsc.parallel_loop(0, D, unroll=4)
            def rcol(d):
                vals = [s_bufs[0][d, pl.ds(g * 16, 16)] for g in range(4)]
                for g in range(4):
                    pv = cbase[g] + ((d - uvec[g]) & 31)
                    plsc.store_scatter(t_bufs[0], [crow[g], pv], vals[g])

            pltpu.sync_copy(
                t_bufs[0].at[pl.ds(0, REM // 4)],
                rm_hbm.at[pl.ds(NBLK * D, REM // 4)],
            )

    return k(tbl_t)


def _sc_gather(idxr, rm):
    mesh = plsc.VectorSubcoreMesh(core_axis_name="c", subcore_axis_name="s")

    @functools.partial(
        pl.kernel,
        mesh=mesh,
        out_type=jax.ShapeDtypeStruct((FIELDS * (D // 8) * BT * 8, 128),
                                      jnp.float32),
        compiler_params=pltpu.CompilerParams(
            use_tc_tiling_on_sc=False, needs_layout_passes=False
        ),
        scratch_types=[
            pltpu.VMEM((NK, 128), jnp.int32),
            pltpu.VMEM((NK, 128), jnp.int32),
            *[pltpu.VMEM((128, D), jnp.float32) for _ in range(NBUF)],
            *[pltpu.VMEM((D, 128), jnp.float32) for _ in range(2)],
            *[pltpu.SemaphoreType.DMA for _ in range(NBUF + 2)],
        ],
    )
    def k(idx_hbm, rm_hbm, out_hbm, idx_v, rem_v, *scr):
        g_bufs = scr[:NBUF]
        t_bufs = scr[NBUF:NBUF + 2]
        g_sems = scr[NBUF + 2:2 * NBUF + 2]
        s_sems = scr[2 * NBUF + 2:]
        wid = lax.axis_index("s") * NC + lax.axis_index("c")
        pltpu.sync_copy(idx_hbm.at[pl.ds(wid * NK, NK)], idx_v)

        @plsc.parallel_loop(0, NK * 8, unroll=4)
        def split(z):
            kc = z // 8
            blk = z - kc * 8
            q = idx_v[kc, pl.ds(blk * 16, 16)]
            rem_v[kc, pl.ds(blk * 16, 16)] = -(q & 31)

        for kk in range(NBUF):
            pltpu.async_copy(rm_hbm.at[idx_v.at[kk]], g_bufs[kk], g_sems[kk])

        rows = [lax.iota(jnp.int32, 16) + blk * 16 for blk in range(8)]

        def body(g, carry):
            for b in range(NBUF):
                t = b % 2
                kc = g * NBUF + b
                bti = kc // FIELDS
                f = kc - bti * FIELDS
                bt = wid * BT_PER_W + bti

                pltpu.make_async_copy(
                    rm_hbm.at[idx_v.at[0]], g_bufs[b], g_sems[b]
                ).wait()

                @pl.when(kc >= 2)
                def _():
                    for dt in range(D // 8):
                        pltpu.make_async_copy(
                            t_bufs[t].at[pl.ds(dt * 8, 8)],
                            out_hbm.at[pl.ds(0, 8)],
                            s_sems[t],
                        ).wait()

                for blk in range(8):
                    remv = rem_v[kc, pl.ds(blk * 16, 16)]

                    @plsc.parallel_loop(0, D, unroll=4)
                    def col(d):
                        v = plsc.load_gather(
                            g_bufs[b], [rows[blk], (remv + d) & 31]
                        )
                        t_bufs[t][d, pl.ds(blk * 16, 16)] = v

                row0 = ((f * (D // 8)) * BT + bt) * 8
                for dt in range(D // 8):
                    pltpu.async_copy(
                        t_bufs[t].at[pl.ds(dt * 8, 8)],
                        out_hbm.at[pl.ds(row0 + dt * BT * 8, 8)],
                        s_sems[t],
                    )

                kn = kc + NBUF

                @pl.when(kn < NK)
                def _():
                    pltpu.async_copy(
                        rm_hbm.at[idx_v.at[kn]], g_bufs[b], g_sems[b]
                    )

            return carry

        lax.fori_loop(0, NK // NBUF, body, 0)

        for t in range(2):
            for dt in range(D // 8):
                pltpu.make_async_copy(
                    t_bufs[t].at[pl.ds(dt * 8, 8)],
                    out_hbm.at[pl.ds(0, 8)],
                    s_sems[t],
                ).wait()

    return k(idxr, rm)


def kernel(inputs, embeddings):
    rm = _sc_transpose(embeddings.T).reshape(VOCAB, D)
    idxr = (
        inputs.reshape(BT, 128, FIELDS)
        .transpose(0, 2, 1)
        .reshape(BT * FIELDS, 128)
    )
    y = _sc_gather(idxr, rm)
    out = y.reshape(FIELDS, D // 8, BT, 8, 128).transpose(2, 4, 0, 1, 3)
    return out.reshape(BATCH, FIELDS, D)
